# Initial kernel scaffold; baseline (speedup 1.0000x reference)
import jax
import jax.numpy as jnp
from jax.experimental import pallas as pl


def kernel(x, Wg, bg, We, be):
    raise NotImplementedError("write your pallas kernel here")


# speedup vs baseline: 1.2714x; 1.2714x over previous
import functools

import jax
import jax.numpy as jnp
from jax.experimental import pallas as pl
from jax.experimental.pallas import tpu as pltpu

TOP_K = 2
NUM_EXPERTS = 8
TOKEN_TILE = 256


def _moe_dense_kernel(x_ref, wg_ref, bg_ref, we_ref, be_ref, out_ref, cw_ref):
    e = pl.program_id(1)
    n_e = pl.num_programs(1)
    x = x_ref[...]

    @pl.when(e == 0)
    def _gating():
        scores = jnp.dot(x, wg_ref[...], preferred_element_type=jnp.float32)
        scores = scores + bg_ref[...][None, :]
        m = jnp.max(scores, axis=-1, keepdims=True)
        ex = jnp.exp(scores - m)
        probs = ex / jnp.sum(ex, axis=-1, keepdims=True)
        lane = jax.lax.broadcasted_iota(jnp.int32, probs.shape, 1)
        i1 = jnp.argmax(probs, axis=-1, keepdims=True)
        mask1 = lane == i1
        neg = jnp.where(mask1, -jnp.inf, probs)
        i2 = jnp.argmax(neg, axis=-1, keepdims=True)
        mask2 = lane == i2
        cw_ref[...] = jnp.where(mask1 | mask2, probs, 0.0)

    cw = cw_ref[...]
    col = jax.lax.broadcasted_iota(jnp.int32, cw.shape, 1)
    w_e = jnp.sum(jnp.where(col == e, cw, 0.0), axis=-1, keepdims=True)
    contrib = w_e * jnp.dot(x, we_ref[0], preferred_element_type=jnp.float32)

    @pl.when(e == 0)
    def _init():
        out_ref[...] = contrib

    @pl.when(e > 0)
    def _acc():
        out_ref[...] += contrib

    @pl.when(e == n_e - 1)
    def _bias():
        out_ref[...] += jnp.dot(cw, be_ref[...], preferred_element_type=jnp.float32)


@jax.jit
def kernel(x, Wg, bg, We, be):
    T, D = x.shape
    E, _, O = We.shape
    grid = (T // TOKEN_TILE, E)
    return pl.pallas_call(
        _moe_dense_kernel,
        grid=grid,
        in_specs=[
            pl.BlockSpec((TOKEN_TILE, D), lambda i, e: (i, 0)),
            pl.BlockSpec((D, E), lambda i, e: (0, 0)),
            pl.BlockSpec((E,), lambda i, e: (0,)),
            pl.BlockSpec((1, D, O), lambda i, e: (e, 0, 0)),
            pl.BlockSpec((E, O), lambda i, e: (0, 0)),
        ],
        out_specs=pl.BlockSpec((TOKEN_TILE, O), lambda i, e: (i, 0)),
        out_shape=jax.ShapeDtypeStruct((T, O), jnp.float32),
        scratch_shapes=[pltpu.VMEM((TOKEN_TILE, E), jnp.float32)],
        compiler_params=pltpu.CompilerParams(
            dimension_semantics=("arbitrary", "arbitrary"),
        ),
    )(x, Wg, bg, We, be)


# speedup vs baseline: 1.2782x; 1.0054x over previous
import functools

import jax
import jax.numpy as jnp
from jax.experimental import pallas as pl
from jax.experimental.pallas import tpu as pltpu

TOP_K = 2
NUM_EXPERTS = 8
TOKEN_TILE = 256


def _moe_dense_kernel(x_ref, wg_ref, bg_ref, we_ref, be_ref, out_ref, cw_ref):
    e = pl.program_id(1)
    n_e = pl.num_programs(1)
    x = x_ref[...]

    @pl.when(e == 0)
    def _gating():
        scores = jnp.dot(x, wg_ref[...], preferred_element_type=jnp.float32)
        scores = scores + bg_ref[...][None, :]
        m = jnp.max(scores, axis=-1, keepdims=True)
        ex = jnp.exp(scores - m)
        probs = ex / jnp.sum(ex, axis=-1, keepdims=True)
        lane = jax.lax.broadcasted_iota(jnp.int32, probs.shape, 1)
        i1 = jnp.argmax(probs, axis=-1, keepdims=True)
        mask1 = lane == i1
        neg = jnp.where(mask1, -jnp.inf, probs)
        i2 = jnp.argmax(neg, axis=-1, keepdims=True)
        mask2 = lane == i2
        cw_ref[...] = jnp.where(mask1 | mask2, probs, 0.0)

    cw = cw_ref[...]
    col = jax.lax.broadcasted_iota(jnp.int32, cw.shape, 1)
    w_e = jnp.sum(jnp.where(col == e, cw, 0.0), axis=-1, keepdims=True)
    contrib = w_e * jnp.dot(
        x.astype(jnp.bfloat16), we_ref[0].astype(jnp.bfloat16),
        preferred_element_type=jnp.float32)

    @pl.when(e == 0)
    def _init():
        out_ref[...] = contrib

    @pl.when(e > 0)
    def _acc():
        out_ref[...] += contrib

    @pl.when(e == n_e - 1)
    def _bias():
        out_ref[...] += jnp.dot(cw, be_ref[...], preferred_element_type=jnp.float32)


@jax.jit
def kernel(x, Wg, bg, We, be):
    T, D = x.shape
    E, _, O = We.shape
    grid = (T // TOKEN_TILE, E)
    return pl.pallas_call(
        _moe_dense_kernel,
        grid=grid,
        in_specs=[
            pl.BlockSpec((TOKEN_TILE, D), lambda i, e: (i, 0)),
            pl.BlockSpec((D, E), lambda i, e: (0, 0)),
            pl.BlockSpec((E,), lambda i, e: (0,)),
            pl.BlockSpec((1, D, O), lambda i, e: (e, 0, 0)),
            pl.BlockSpec((E, O), lambda i, e: (0, 0)),
        ],
        out_specs=pl.BlockSpec((TOKEN_TILE, O), lambda i, e: (i, 0)),
        out_shape=jax.ShapeDtypeStruct((T, O), jnp.float32),
        scratch_shapes=[pltpu.VMEM((TOKEN_TILE, E), jnp.float32)],
        compiler_params=pltpu.CompilerParams(
            dimension_semantics=("arbitrary", "arbitrary"),
        ),
    )(x, Wg, bg, We, be)


# speedup vs baseline: 2.5551x; 1.9989x over previous
import functools

import jax
import jax.numpy as jnp
from jax.experimental import pallas as pl
from jax.experimental.pallas import tpu as pltpu

TOP_K = 2
NUM_EXPERTS = 8
TOKEN_TILE = 256


def _moe_dense_kernel(x_ref, wg_ref, bg_ref, we_ref, be_ref, out_ref):
    x = x_ref[...]
    scores = jnp.dot(x, wg_ref[...], preferred_element_type=jnp.float32)
    scores = scores + bg_ref[...][None, :]
    m = jnp.max(scores, axis=-1, keepdims=True)
    ex = jnp.exp(scores - m)
    probs = ex / jnp.sum(ex, axis=-1, keepdims=True)
    lane = jax.lax.broadcasted_iota(jnp.int32, probs.shape, 1)
    i1 = jnp.argmax(probs, axis=-1, keepdims=True)
    mask1 = lane == i1
    neg = jnp.where(mask1, -jnp.inf, probs)
    i2 = jnp.argmax(neg, axis=-1, keepdims=True)
    mask2 = lane == i2
    cw = jnp.where(mask1 | mask2, probs, 0.0)

    xb = x.astype(jnp.bfloat16)
    acc = jnp.dot(cw, be_ref[...], preferred_element_type=jnp.float32)
    for e in range(NUM_EXPERTS):
        y = jnp.dot(xb, we_ref[e], preferred_element_type=jnp.float32)
        acc = acc + cw[:, e:e + 1] * y
    out_ref[...] = acc


@jax.jit
def kernel(x, Wg, bg, We, be):
    T, D = x.shape
    E, _, O = We.shape
    We_b = We.astype(jnp.bfloat16)
    grid = (T // TOKEN_TILE,)
    return pl.pallas_call(
        _moe_dense_kernel,
        grid=grid,
        in_specs=[
            pl.BlockSpec((TOKEN_TILE, D), lambda i: (i, 0)),
            pl.BlockSpec((D, E), lambda i: (0, 0)),
            pl.BlockSpec((E,), lambda i: (0,)),
            pl.BlockSpec((E, D, O), lambda i: (0, 0, 0)),
            pl.BlockSpec((E, O), lambda i: (0, 0)),
        ],
        out_specs=pl.BlockSpec((TOKEN_TILE, O), lambda i: (i, 0)),
        out_shape=jax.ShapeDtypeStruct((T, O), jnp.float32),
        compiler_params=pltpu.CompilerParams(
            dimension_semantics=("arbitrary",),
        ),
    )(x, Wg, bg, We_b, be)
